# Initial kernel scaffold; baseline (speedup 1.0000x reference)
import jax
import jax.numpy as jnp
from jax.experimental import pallas as pl


def kernel(x, W):
    raise NotImplementedError("write your pallas kernel here")


# speedup vs baseline: 4.7504x; 4.7504x over previous
import jax
import jax.numpy as jnp
from jax import lax
from jax.experimental import pallas as pl
from jax.experimental.pallas import tpu as pltpu
from jax.experimental.pallas import tpu_sc as plsc

NC = 2
NS = 16
NW = NC * NS

B_ROWS = 16384
B_COLS = 200
D = 64
B = B_ROWS * B_COLS
PER_W = B // NW
SUB = 4
CHUNK = SUB * 128
NCHUNK = PER_W // CHUNK
IDX_ROWS_PER_W = PER_W // 128


def _body(x_hbm, w_hbm, out_hbm, idx_v, rows_v, sem):
    c = lax.axis_index("c")
    s = lax.axis_index("s")
    wid = s * NC + c
    row0 = wid * PER_W
    irow0 = wid * IDX_ROWS_PER_W

    def chunk(i, carry):
        pltpu.sync_copy(x_hbm.at[pl.ds(irow0 + i * SUB, SUB)], idx_v)
        cps = [
            pltpu.async_copy(
                w_hbm.at[idx_v.at[j]], rows_v.at[pl.ds(j * 128, 128)], sem
            )
            for j in range(SUB)
        ]
        for cp in cps:
            cp.wait()
        pltpu.sync_copy(rows_v, out_hbm.at[pl.ds(row0 + i * CHUNK, CHUNK)])
        return carry

    lax.fori_loop(0, NCHUNK, chunk, 0)


@jax.jit
def kernel(x, W):
    xf = x.astype(jnp.int32).reshape(B // 128, 128)
    mesh = plsc.VectorSubcoreMesh(
        core_axis_name="c", subcore_axis_name="s", num_cores=NC, num_subcores=NS
    )
    out = pl.kernel(
        _body,
        out_type=jax.ShapeDtypeStruct((B, D), jnp.float32),
        mesh=mesh,
        scratch_types=[
            pltpu.VMEM((SUB, 128), jnp.int32),
            pltpu.VMEM((CHUNK, D), jnp.float32),
            pltpu.SemaphoreType.DMA,
        ],
        compiler_params=pltpu.CompilerParams(use_tc_tiling_on_sc=False),
    )(xf, W)
    return out.reshape(B_ROWS, B_COLS, D)


# speedup vs baseline: 5.1684x; 1.0880x over previous
import jax
import jax.numpy as jnp
from jax import lax
from jax.experimental import pallas as pl
from jax.experimental.pallas import tpu as pltpu
from jax.experimental.pallas import tpu_sc as plsc

NC = 2
NS = 16
NW = NC * NS

B_ROWS = 16384
B_COLS = 200
D = 64
B = B_ROWS * B_COLS
PER_W = B // NW
SUB = 4
CHUNK = SUB * 128
NCHUNK = PER_W // CHUNK
IDX_ROWS_PER_W = PER_W // 128

CH_PER_SUPER = 8
IDXROWS_SUPER = CH_PER_SUPER * SUB
NSUPER = NCHUNK // CH_PER_SUPER


def _body(x_hbm, w_hbm, out_hbm, idx_v, rows_v, gsem, ssem0, ssem1, isem):
    c = lax.axis_index("c")
    s_ax = lax.axis_index("s")
    wid = s_ax * NC + c
    row0 = wid * PER_W
    irow0 = wid * IDX_ROWS_PER_W

    pltpu.sync_copy(x_hbm.at[pl.ds(irow0, IDXROWS_SUPER)], idx_v.at[0])

    def souter(sp, carry):
        slot = lax.rem(sp, 2)

        @pl.when(sp > 0)
        def _():
            pltpu.make_async_copy(
                x_hbm.at[pl.ds(irow0, IDXROWS_SUPER)], idx_v.at[slot], isem
            ).wait()

        @pl.when(sp < NSUPER - 1)
        def _():
            pltpu.async_copy(
                x_hbm.at[pl.ds(irow0 + (sp + 1) * IDXROWS_SUPER, IDXROWS_SUPER)],
                idx_v.at[1 - slot],
                isem,
            )

        for j in range(CH_PER_SUPER):
            b = j % 2
            ssem = ssem0 if b == 0 else ssem1
            if j >= 2:
                pltpu.make_async_copy(
                    rows_v.at[b], out_hbm.at[pl.ds(row0, CHUNK)], ssem
                ).wait()
            else:

                @pl.when(sp > 0)
                def _(b=b, ssem=ssem):
                    pltpu.make_async_copy(
                        rows_v.at[b], out_hbm.at[pl.ds(row0, CHUNK)], ssem
                    ).wait()

            cps = [
                pltpu.async_copy(
                    w_hbm.at[idx_v.at[slot, j * SUB + jj]],
                    rows_v.at[b, pl.ds(jj * 128, 128)],
                    gsem,
                )
                for jj in range(SUB)
            ]
            for cp in cps:
                cp.wait()
            pltpu.async_copy(
                rows_v.at[b],
                out_hbm.at[pl.ds(row0 + (sp * CH_PER_SUPER + j) * CHUNK, CHUNK)],
                ssem,
            )
        return carry

    lax.fori_loop(0, NSUPER, souter, 0)
    pltpu.make_async_copy(rows_v.at[0], out_hbm.at[pl.ds(row0, CHUNK)], ssem0).wait()
    pltpu.make_async_copy(rows_v.at[1], out_hbm.at[pl.ds(row0, CHUNK)], ssem1).wait()


@jax.jit
def kernel(x, W):
    xf = x.astype(jnp.int32).reshape(B // 128, 128)
    mesh = plsc.VectorSubcoreMesh(
        core_axis_name="c", subcore_axis_name="s", num_cores=NC, num_subcores=NS
    )
    out = pl.kernel(
        _body,
        out_type=jax.ShapeDtypeStruct((B, D), jnp.float32),
        mesh=mesh,
        scratch_types=[
            pltpu.VMEM((2, IDXROWS_SUPER, 128), jnp.int32),
            pltpu.VMEM((2, CHUNK, D), jnp.float32),
            pltpu.SemaphoreType.DMA,
            pltpu.SemaphoreType.DMA,
            pltpu.SemaphoreType.DMA,
            pltpu.SemaphoreType.DMA,
        ],
        compiler_params=pltpu.CompilerParams(use_tc_tiling_on_sc=False),
    )(xf, W)
    return out.reshape(B_ROWS, B_COLS, D)


# speedup vs baseline: 5.1716x; 1.0006x over previous
import jax
import jax.numpy as jnp
from jax import lax
from jax.experimental import pallas as pl
from jax.experimental.pallas import tpu as pltpu
from jax.experimental.pallas import tpu_sc as plsc

NC = 2
NS = 16
NW = NC * NS

B_ROWS = 16384
B_COLS = 200
D = 64
B = B_ROWS * B_COLS
PER_W = B // NW
SUB = 4
CHUNK = SUB * 128
NCHUNK = PER_W // CHUNK
IDX_ROWS_PER_W = PER_W // 128

CH_PER_SUPER = 8
IDXROWS_SUPER = CH_PER_SUPER * SUB
NSUPER = NCHUNK // CH_PER_SUPER


def _body(x_hbm, w_hbm, out_hbm, idx_v, rows_v, gsem0, gsem1, ssem0, ssem1, isem):
    c = lax.axis_index("c")
    s_ax = lax.axis_index("s")
    wid = s_ax * NC + c
    row0 = wid * PER_W
    irow0 = wid * IDX_ROWS_PER_W

    gsems = (gsem0, gsem1)
    ssems = (ssem0, ssem1)

    pltpu.sync_copy(x_hbm.at[pl.ds(irow0, IDXROWS_SUPER)], idx_v.at[0])

    def fire_gathers(slot, j, b):
        for jj in range(SUB):
            pltpu.async_copy(
                w_hbm.at[idx_v.at[slot, j * SUB + jj]],
                rows_v.at[b, pl.ds(jj * 128, 128)],
                gsems[b],
            )

    def drain_gathers(b):
        for jj in range(SUB):
            pltpu.make_async_copy(
                w_hbm.at[idx_v.at[0, jj]],
                rows_v.at[b, pl.ds(jj * 128, 128)],
                gsems[b],
            ).wait()

    def issue_store(b, i):
        pltpu.async_copy(
            rows_v.at[b], out_hbm.at[pl.ds(row0 + i * CHUNK, CHUNK)], ssems[b]
        )

    def drain_store(b):
        pltpu.make_async_copy(
            rows_v.at[b], out_hbm.at[pl.ds(row0, CHUNK)], ssems[b]
        ).wait()

    def souter(sp, carry):
        slot = lax.rem(sp, 2)

        @pl.when(sp > 0)
        def _():
            pltpu.make_async_copy(
                x_hbm.at[pl.ds(irow0, IDXROWS_SUPER)], idx_v.at[slot], isem
            ).wait()

        for j in range(CH_PER_SUPER):
            b = j % 2
            i = sp * CH_PER_SUPER + j
            if j >= 2:
                drain_store(b)
            else:

                @pl.when(sp > 0)
                def _(b=b):
                    drain_store(b)

            fire_gathers(slot, j, b)
            if j >= 1:
                drain_gathers(1 - b)
                issue_store(1 - b, i - 1)
            else:

                @pl.when(sp > 0)
                def _(b=b, i=i):
                    drain_gathers(1 - b)
                    issue_store(1 - b, i - 1)

            if j == 0:
                @pl.when(sp < NSUPER - 1)
                def _(slot=slot):
                    pltpu.async_copy(
                        x_hbm.at[
                            pl.ds(irow0 + (sp + 1) * IDXROWS_SUPER, IDXROWS_SUPER)
                        ],
                        idx_v.at[1 - slot],
                        isem,
                    )
        return carry

    lax.fori_loop(0, NSUPER, souter, 0)
    last = NCHUNK - 1
    bl = last % 2
    drain_gathers(bl)
    issue_store(bl, last)
    drain_store(1 - bl)
    drain_store(bl)


@jax.jit
def kernel(x, W):
    xf = x.astype(jnp.int32).reshape(B // 128, 128)
    mesh = plsc.VectorSubcoreMesh(
        core_axis_name="c", subcore_axis_name="s", num_cores=NC, num_subcores=NS
    )
    out = pl.kernel(
        _body,
        out_type=jax.ShapeDtypeStruct((B, D), jnp.float32),
        mesh=mesh,
        scratch_types=[
            pltpu.VMEM((2, IDXROWS_SUPER, 128), jnp.int32),
            pltpu.VMEM((2, CHUNK, D), jnp.float32),
            pltpu.SemaphoreType.DMA,
            pltpu.SemaphoreType.DMA,
            pltpu.SemaphoreType.DMA,
            pltpu.SemaphoreType.DMA,
            pltpu.SemaphoreType.DMA,
        ],
        compiler_params=pltpu.CompilerParams(use_tc_tiling_on_sc=False),
    )(xf, W)
    return out.reshape(B_ROWS, B_COLS, D)
